# baseline (speedup 1.0000x reference)
import functools

import jax
import jax.numpy as jnp
from jax import lax
from jax.experimental import pallas as pl
from jax.experimental.pallas import tpu as pltpu
from jax.experimental.pallas import tpu_sc as plsc

_NUM_CORES = 2
_NUM_SUBCORES = 16
_NUM_WORKERS = _NUM_CORES * _NUM_SUBCORES

_CHUNK = 1024
_LANES = 16
_ZCOLS = 1920


def _transpose_table(z_t):
    d, n_rows = z_t.shape
    n_full = n_rows // _ZCOLS
    full_span = n_full * _ZCOLS
    tails = ((full_span, n_rows - full_span),)
    n_t = (n_full + _NUM_WORKERS - 1) // _NUM_WORKERS

    mesh = plsc.VectorSubcoreMesh(core_axis_name="c", subcore_axis_name="s")

    @functools.partial(
        pl.kernel,
        mesh=mesh,
        out_type=jax.ShapeDtypeStruct((n_rows * d,), jnp.float32),
        scratch_types=(
            [pltpu.VMEM((d, _ZCOLS + 1), jnp.float32) for _ in range(2)]
            + [pltpu.VMEM((_ZCOLS * d,), jnp.float32) for _ in range(2)]
            + [pltpu.SemaphoreType.DMA for _ in range(4)]
        ),
        compiler_params=pltpu.CompilerParams(
            use_tc_tiling_on_sc=True, needs_layout_passes=False,
            disable_bounds_checks=True),
    )
    def tk(zt_hbm, out_hbm, zi0, zi1, zo0, zo1, li0, li1, so0, so1):
        zin = (zi0, zi1)
        zout = (zo0, zo1)
        sem_l = (li0, li1)
        sem_s = (so0, so1)

        wid = lax.axis_index("s") * _NUM_CORES + lax.axis_index("c")
        lane = lax.iota(jnp.int32, _LANES)

        def load_copy(b, col0, cols):
            return pltpu.make_async_copy(
                zt_hbm.at[:, pl.ds(col0, cols)],
                zin[b].at[:, pl.ds(0, cols)], sem_l[b])

        def store_copy(b, col0, cols):
            return pltpu.make_async_copy(
                zout[b].at[pl.ds(0, cols * d)],
                out_hbm.at[pl.ds(col0 * d, cols * d)], sem_s[b])

        def transpose_block(b, cols):
            def body(ig, carry):
                i0 = ig * _LANES
                rows16 = [
                    plsc.load_gather(zin[b], [lane, lane * 0 + (i0 + u)])
                    for u in range(_LANES)
                ]
                for u in range(_LANES):
                    zout[b][pl.ds((i0 + u) * d, d)] = rows16[u]
                return carry
            lax.fori_loop(0, cols // _LANES, body, 0)

        load_copy(0, wid * _ZCOLS, _ZCOLS).start()
        for t in range(n_t):
            c = wid + t * _NUM_WORKERS
            b = t % 2

            @pl.when(c < n_full)
            def _():
                nxt = c + _NUM_WORKERS

                @pl.when(nxt < n_full)
                def _():
                    load_copy(1 - b, nxt * _ZCOLS, _ZCOLS).start()

                if t >= 2:
                    store_copy(b, 0, _ZCOLS).wait()
                load_copy(b, 0, _ZCOLS).wait()
                transpose_block(b, _ZCOLS)
                store_copy(b, c * _ZCOLS, _ZCOLS).start()

        @pl.when(wid == _NUM_WORKERS - 1)
        def _():
            store_copy(0, 0, _ZCOLS).wait()
            for col0, cols in tails:
                pltpu.sync_copy(zt_hbm.at[:, pl.ds(col0, cols)],
                                zin[0].at[:, pl.ds(0, cols)])
                transpose_block(0, cols)
                pltpu.sync_copy(zout[0].at[pl.ds(0, cols * d)],
                                out_hbm.at[pl.ds(col0 * d, cols * d)])
            store_copy(1, 0, _ZCOLS).wait()

        @pl.when(wid < _NUM_WORKERS - 1)
        def _():
            for b in range(2):
                store_copy(b, 0, _ZCOLS).wait()

    return tk(z_t)


def _gather(table, idx_t):
    h_len, p_len = idx_t.shape
    d = table.shape[1]
    n_chunks = p_len // _CHUNK
    n_slabs_max = (h_len + _NUM_WORKERS - 1) // _NUM_WORKERS
    slab_stride = (d // 8) * p_len * 8
    db_stride = p_len * 8
    chunk_out = _CHUNK * d
    half_chunk = chunk_out // 2

    mesh = plsc.VectorSubcoreMesh(core_axis_name="c", subcore_axis_name="s")

    @functools.partial(
        pl.kernel,
        mesh=mesh,
        out_type=jax.ShapeDtypeStruct((p_len * h_len * d,), jnp.float32),
        scratch_types=(
            [pltpu.VMEM((p_len,), jnp.int32)]
            + [pltpu.VMEM((_CHUNK, d), jnp.float32) for _ in range(2)]
            + [pltpu.VMEM((chunk_out,), jnp.float32) for _ in range(2)]
            + [pltpu.SemaphoreType.DMA for _ in range(4)]
        ),
        compiler_params=pltpu.CompilerParams(
            use_tc_tiling_on_sc=False, needs_layout_passes=False,
            disable_bounds_checks=True),
    )
    def k(table_hbm, idx_hbm, out_hbm, idx_s, r0, r1, t0, t1, gs0, gs1,
          ss0, ss1):
        rows = (r0, r1)
        trans = (t0, t1)
        sem_g = (gs0, gs1)
        sem_s = (ss0, ss1)

        wid = lax.axis_index("s") * _NUM_CORES + lax.axis_index("c")
        lane = lax.iota(jnp.int32, _LANES)

        def transpose_chunk(b):
            def body(pg, carry):
                pvec = pg * 16 + lane
                scal = (pg // 8) * 1024 + lax.rem(pg, 8) * 16
                cols = [
                    plsc.load_gather(rows[b], [pvec, lane * 0 + dd])
                    for dd in range(d)
                ]
                for dd in range(d):
                    off = (dd // 8) * half_chunk + (dd % 8) * 128
                    trans[b][pl.ds(off + scal, 16)] = cols[dd]
                return carry
            lax.fori_loop(0, _CHUNK // 16, body, 0, unroll=2)

        def gather_copy(b, pj):
            return pltpu.make_async_copy(
                table_hbm.at[idx_s.at[pl.ds(pj * _CHUNK, _CHUNK)]],
                rows[b], sem_g[b])

        def store_copy(b, h, pj, db):
            dst = out_hbm.at[pl.ds(
                h * slab_stride + db * db_stride + pj * half_chunk,
                half_chunk)]
            return pltpu.make_async_copy(
                trans[b].at[pl.ds(db * half_chunk, half_chunk)], dst,
                sem_s[b])

        for t in range(n_slabs_max):
            h = wid + t * _NUM_WORKERS

            @pl.when(h < h_len)
            def _():
                pltpu.sync_copy(idx_hbm.at[h], idx_s)
                gather_copy(0, 0).start()

                def chunk_group(g, carry):
                    for b in range(2):
                        pj = g * 2 + b

                        @pl.when(pj + 1 < n_chunks)
                        def _():
                            gather_copy(1 - b, pj + 1).start()

                        gather_copy(b, pj).wait()

                        @pl.when(t * n_chunks + pj >= 2)
                        def _():
                            for db in range(2):
                                store_copy(b, 0, 0, db).wait()

                        transpose_chunk(b)
                        for db in range(2):
                            store_copy(b, h, pj, db).start()
                    return carry

                lax.fori_loop(0, n_chunks // 2, chunk_group, 0)

        for b in range(2):
            for db in range(2):
                store_copy(b, 0, 0, db).wait()

    return k(table, idx_t)


def kernel(z, idx):
    p, h = idx.shape
    d = z.shape[1]
    idx_t = jnp.transpose(idx.astype(jnp.int32))
    z_pad = jnp.pad(z, ((0, (-z.shape[0]) % 128), (0, 0)))
    table = _transpose_table(jnp.transpose(z_pad)).reshape(-1, d)
    out_flat = _gather(table, idx_t)
    out5d = out_flat.reshape(h, d // 8, p // 128, 8, 128)
    return out5d.transpose(2, 4, 0, 1, 3).reshape(p, h, d)


# speedup vs baseline: 1.0051x; 1.0051x over previous
import functools

import jax
import jax.numpy as jnp
from jax import lax
from jax.experimental import pallas as pl
from jax.experimental.pallas import tpu as pltpu
from jax.experimental.pallas import tpu_sc as plsc

_NUM_CORES = 2
_NUM_SUBCORES = 16
_NUM_WORKERS = _NUM_CORES * _NUM_SUBCORES

_CHUNK = 1024
_LANES = 16
_ZCOLS = 1920


def _transpose_table(z_t):
    d, n_rows = z_t.shape
    n_full = n_rows // _ZCOLS
    full_span = n_full * _ZCOLS
    tails = ((full_span, n_rows - full_span),)
    n_t = (n_full + _NUM_WORKERS - 1) // _NUM_WORKERS

    mesh = plsc.VectorSubcoreMesh(core_axis_name="c", subcore_axis_name="s")

    @functools.partial(
        pl.kernel,
        mesh=mesh,
        out_type=jax.ShapeDtypeStruct((n_rows * d,), jnp.float32),
        scratch_types=(
            [pltpu.VMEM((d, _ZCOLS), jnp.float32) for _ in range(2)]
            + [pltpu.VMEM((_ZCOLS * d,), jnp.float32) for _ in range(2)]
            + [pltpu.SemaphoreType.DMA for _ in range(4)]
        ),
        compiler_params=pltpu.CompilerParams(
            use_tc_tiling_on_sc=True, needs_layout_passes=False,
            disable_bounds_checks=True),
    )
    def tk(zt_hbm, out_hbm, zi0, zi1, zo0, zo1, li0, li1, so0, so1):
        zin = (zi0, zi1)
        zout = (zo0, zo1)
        sem_l = (li0, li1)
        sem_s = (so0, so1)

        wid = lax.axis_index("s") * _NUM_CORES + lax.axis_index("c")
        lane = lax.iota(jnp.int32, _LANES)

        def load_copy(b, col0, cols):
            return pltpu.make_async_copy(
                zt_hbm.at[:, pl.ds(col0, cols)],
                zin[b].at[:, pl.ds(0, cols)], sem_l[b])

        def store_copy(b, col0, cols):
            return pltpu.make_async_copy(
                zout[b].at[pl.ds(0, cols * d)],
                out_hbm.at[pl.ds(col0 * d, cols * d)], sem_s[b])

        def transpose_block(b, cols):
            def body(ig, carry):
                i0 = ig * _LANES
                rows16 = [
                    plsc.load_gather(zin[b], [lane, lane * 0 + (i0 + u)])
                    for u in range(_LANES)
                ]
                for u in range(_LANES):
                    zout[b][pl.ds((i0 + u) * d, d)] = rows16[u]
                return carry
            lax.fori_loop(0, cols // _LANES, body, 0)

        load_copy(0, wid * _ZCOLS, _ZCOLS).start()
        for t in range(n_t):
            c = wid + t * _NUM_WORKERS
            b = t % 2

            @pl.when(c < n_full)
            def _():
                nxt = c + _NUM_WORKERS

                @pl.when(nxt < n_full)
                def _():
                    load_copy(1 - b, nxt * _ZCOLS, _ZCOLS).start()

                if t >= 2:
                    store_copy(b, 0, _ZCOLS).wait()
                load_copy(b, 0, _ZCOLS).wait()
                transpose_block(b, _ZCOLS)
                store_copy(b, c * _ZCOLS, _ZCOLS).start()

        @pl.when(wid == _NUM_WORKERS - 1)
        def _():
            store_copy(0, 0, _ZCOLS).wait()
            for col0, cols in tails:
                pltpu.sync_copy(zt_hbm.at[:, pl.ds(col0, cols)],
                                zin[0].at[:, pl.ds(0, cols)])
                transpose_block(0, cols)
                pltpu.sync_copy(zout[0].at[pl.ds(0, cols * d)],
                                out_hbm.at[pl.ds(col0 * d, cols * d)])
            store_copy(1, 0, _ZCOLS).wait()

        @pl.when(wid < _NUM_WORKERS - 1)
        def _():
            for b in range(2):
                store_copy(b, 0, _ZCOLS).wait()

    return tk(z_t)


def _gather(table, idx_t):
    h_len, p_len = idx_t.shape
    d = table.shape[1]
    n_chunks = p_len // _CHUNK
    n_slabs_max = (h_len + _NUM_WORKERS - 1) // _NUM_WORKERS
    slab_stride = (d // 8) * p_len * 8
    db_stride = p_len * 8
    chunk_out = _CHUNK * d
    half_chunk = chunk_out // 2

    mesh = plsc.VectorSubcoreMesh(core_axis_name="c", subcore_axis_name="s")

    @functools.partial(
        pl.kernel,
        mesh=mesh,
        out_type=jax.ShapeDtypeStruct((p_len * h_len * d,), jnp.float32),
        scratch_types=(
            [pltpu.VMEM((p_len,), jnp.int32)]
            + [pltpu.VMEM((_CHUNK, d), jnp.float32) for _ in range(2)]
            + [pltpu.VMEM((chunk_out,), jnp.float32) for _ in range(2)]
            + [pltpu.SemaphoreType.DMA for _ in range(4)]
        ),
        compiler_params=pltpu.CompilerParams(
            use_tc_tiling_on_sc=False, needs_layout_passes=False,
            disable_bounds_checks=True),
    )
    def k(table_hbm, idx_hbm, out_hbm, idx_s, r0, r1, t0, t1, gs0, gs1,
          ss0, ss1):
        rows = (r0, r1)
        trans = (t0, t1)
        sem_g = (gs0, gs1)
        sem_s = (ss0, ss1)

        wid = lax.axis_index("s") * _NUM_CORES + lax.axis_index("c")
        lane = lax.iota(jnp.int32, _LANES)

        def transpose_chunk(b):
            def body(pg, carry):
                pvec = pg * 16 + lane
                scal = (pg // 8) * 1024 + lax.rem(pg, 8) * 16
                cols = [
                    plsc.load_gather(rows[b], [pvec, lane * 0 + dd])
                    for dd in range(d)
                ]
                for dd in range(d):
                    off = (dd // 8) * half_chunk + (dd % 8) * 128
                    trans[b][pl.ds(off + scal, 16)] = cols[dd]
                return carry
            lax.fori_loop(0, _CHUNK // 16, body, 0, unroll=3)

        def gather_copy(b, pj):
            return pltpu.make_async_copy(
                table_hbm.at[idx_s.at[pl.ds(pj * _CHUNK, _CHUNK)]],
                rows[b], sem_g[b])

        def store_copy(b, h, pj, db):
            dst = out_hbm.at[pl.ds(
                h * slab_stride + db * db_stride + pj * half_chunk,
                half_chunk)]
            return pltpu.make_async_copy(
                trans[b].at[pl.ds(db * half_chunk, half_chunk)], dst,
                sem_s[b])

        for t in range(n_slabs_max):
            h = wid + t * _NUM_WORKERS

            @pl.when(h < h_len)
            def _():
                pltpu.sync_copy(idx_hbm.at[h], idx_s)
                gather_copy(0, 0).start()

                def chunk_group(g, carry):
                    for b in range(2):
                        pj = g * 2 + b

                        @pl.when(pj + 1 < n_chunks)
                        def _():
                            gather_copy(1 - b, pj + 1).start()

                        gather_copy(b, pj).wait()

                        @pl.when(t * n_chunks + pj >= 2)
                        def _():
                            for db in range(2):
                                store_copy(b, 0, 0, db).wait()

                        transpose_chunk(b)
                        for db in range(2):
                            store_copy(b, h, pj, db).start()
                    return carry

                lax.fori_loop(0, n_chunks // 2, chunk_group, 0)

        for b in range(2):
            for db in range(2):
                store_copy(b, 0, 0, db).wait()

    return k(table, idx_t)


def kernel(z, idx):
    p, h = idx.shape
    d = z.shape[1]
    idx_t = jnp.transpose(idx.astype(jnp.int32))
    z_pad = jnp.pad(z, ((0, (-z.shape[0]) % 128), (0, 0)))
    table = _transpose_table(jnp.transpose(z_pad)).reshape(-1, d)
    out_flat = _gather(table, idx_t)
    out5d = out_flat.reshape(h, d // 8, p // 128, 8, 128)
    return out5d.transpose(2, 4, 0, 1, 3).reshape(p, h, d)
